# Initial kernel scaffold; baseline (speedup 1.0000x reference)
import jax
import jax.numpy as jnp
from jax.experimental import pallas as pl


def kernel(species_tokens, ability_tokens, item_tokens, move_tokens, num_moves, species_table, items_table, abilities_table, moves_table, pre_species, pre_items, pre_abilities, pre_moves, species_W, items_W, abilities_W, moves_W):
    raise NotImplementedError("write your pallas kernel here")


# speedup vs baseline: 1.7763x; 1.7763x over previous
import functools

import jax
import jax.numpy as jnp
from jax import lax
from jax.experimental import pallas as pl
from jax.experimental.pallas import tpu as pltpu
from jax.experimental.pallas import tpu_sc as plsc

_NC, _NS = 2, 16
_NW = _NC * _NS
_SUB = 64


def _accum3(acc, b, c, nrows, ncol):
    def body(r, carry):
        for g in range(ncol // 16):
            sl = pl.ds(g * 16, 16)
            acc[r, sl] = acc[r, sl] + b[r, sl] + c[r, sl]
        return carry
    lax.fori_loop(0, nrows, body, 0)


def _reduce4(dst, src, nrows_out, ncol):
    def body(t, carry):
        for g in range(ncol // 16):
            sl = pl.ds(g * 16, 16)
            dst[t, sl] = (src[4 * t, sl] + src[4 * t + 1, sl]
                          + src[4 * t + 2, sl] + src[4 * t + 3, sl])
        return carry
    lax.fori_loop(0, nrows_out, body, 0)


def _sc_gather_stage(s_tok, i_tok, a_tok, m_flat,
                     species_table, items_table, abilities_table, moves_table,
                     pre_species, pre_items, pre_abilities, pre_moves):
    B = s_tok.shape[0]
    D = species_table.shape[1]
    P = pre_species.shape[1]
    f32 = jnp.float32
    chunk = B // _NW
    nstep = chunk // _SUB

    mesh = plsc.VectorSubcoreMesh(core_axis_name="c", subcore_axis_name="s",
                                  num_cores=_NC, num_subcores=_NS)

    @functools.partial(
        pl.kernel,
        out_type=(
            jax.ShapeDtypeStruct((B, D), f32),
            jax.ShapeDtypeStruct((B, D), f32),
            jax.ShapeDtypeStruct((B, P), f32),
            jax.ShapeDtypeStruct((B, P), f32),
            jax.ShapeDtypeStruct((B, P), f32),
            jax.ShapeDtypeStruct((B, P), f32),
        ),
        mesh=mesh,
        compiler_params=pltpu.CompilerParams(use_tc_tiling_on_sc=False),
        scratch_types=[
            pltpu.VMEM((_SUB,), jnp.int32),
            pltpu.VMEM((_SUB,), jnp.int32),
            pltpu.VMEM((_SUB,), jnp.int32),
            pltpu.VMEM((2, 2 * _SUB), jnp.int32),
            pltpu.VMEM((_SUB, D), f32),
            pltpu.VMEM((_SUB, D), f32),
            pltpu.VMEM((_SUB, D), f32),
            pltpu.VMEM((4 * _SUB, D), f32),
            pltpu.VMEM((_SUB, D), f32),
            pltpu.VMEM((_SUB, P), f32),
            pltpu.VMEM((_SUB, P), f32),
            pltpu.VMEM((_SUB, P), f32),
            pltpu.VMEM((4 * _SUB, P), f32),
            pltpu.VMEM((_SUB, P), f32),
            pltpu.SemaphoreType.DMA,
            pltpu.SemaphoreType.DMA,
            pltpu.SemaphoreType.DMA,
            pltpu.SemaphoreType.DMA,
            pltpu.SemaphoreType.DMA,
            pltpu.SemaphoreType.DMA,
            pltpu.SemaphoreType.DMA,
            pltpu.SemaphoreType.DMA,
        ],
    )
    def sc_k(s_hbm, i_hbm, a_hbm, m_hbm,
             st_hbm, it_hbm, at_hbm, mt_hbm,
             ps_hbm, pi_hbm, pa_hbm, pm_hbm,
             embsum_hbm, msum_hbm, preS_hbm, preI_hbm, preA_hbm, pmsum_hbm,
             sidx, iidx, aidx, midx, rowsS, rowsI, rowsA, mrows, msumb,
             prebufS, prebufI, prebufA, pmrows, pmsumb,
             semS, semI, semA, semM, semPS, semPI, semPA, semPM):
        wid = lax.axis_index("s") * _NC + lax.axis_index("c")
        tile_base = wid * chunk

        def step_body(step, carry):
            base = tile_base + step * _SUB
            pltpu.sync_copy(s_hbm.at[pl.ds(base, _SUB)], sidx)
            pltpu.sync_copy(i_hbm.at[pl.ds(base, _SUB)], iidx)
            pltpu.sync_copy(a_hbm.at[pl.ds(base, _SUB)], aidx)
            pltpu.sync_copy(m_hbm.at[pl.ds(4 * base, 2 * _SUB)], midx.at[0])
            pltpu.sync_copy(m_hbm.at[pl.ds(4 * base + 2 * _SUB, 2 * _SUB)],
                            midx.at[1])

            cS = pltpu.async_copy(st_hbm.at[sidx], rowsS, semS)
            cI = pltpu.async_copy(it_hbm.at[iidx], rowsI, semI)
            cA = pltpu.async_copy(at_hbm.at[aidx], rowsA, semA)
            cM0 = pltpu.async_copy(mt_hbm.at[midx.at[0]],
                                   mrows.at[pl.ds(0, 2 * _SUB)], semM)
            cM1 = pltpu.async_copy(mt_hbm.at[midx.at[1]],
                                   mrows.at[pl.ds(2 * _SUB, 2 * _SUB)], semM)
            cPS = pltpu.async_copy(ps_hbm.at[sidx], prebufS, semPS)
            cPI = pltpu.async_copy(pi_hbm.at[iidx], prebufI, semPI)
            cPA = pltpu.async_copy(pa_hbm.at[aidx], prebufA, semPA)
            cPM0 = pltpu.async_copy(pm_hbm.at[midx.at[0]],
                                    pmrows.at[pl.ds(0, 2 * _SUB)], semPM)
            cPM1 = pltpu.async_copy(pm_hbm.at[midx.at[1]],
                                    pmrows.at[pl.ds(2 * _SUB, 2 * _SUB)], semPM)

            cS.wait()
            cI.wait()
            cA.wait()
            _accum3(rowsS, rowsI, rowsA, _SUB, D)
            pltpu.sync_copy(rowsS, embsum_hbm.at[pl.ds(base, _SUB)])

            cM0.wait()
            cM1.wait()
            _reduce4(msumb, mrows, _SUB, D)
            pltpu.sync_copy(msumb, msum_hbm.at[pl.ds(base, _SUB)])

            cPS.wait()
            pltpu.sync_copy(prebufS, preS_hbm.at[pl.ds(base, _SUB)])
            cPI.wait()
            pltpu.sync_copy(prebufI, preI_hbm.at[pl.ds(base, _SUB)])
            cPA.wait()
            pltpu.sync_copy(prebufA, preA_hbm.at[pl.ds(base, _SUB)])

            cPM0.wait()
            cPM1.wait()
            _reduce4(pmsumb, pmrows, _SUB, P)
            pltpu.sync_copy(pmsumb, pmsum_hbm.at[pl.ds(base, _SUB)])
            return carry

        lax.fori_loop(0, nstep, step_body, 0)

    return sc_k(s_tok, i_tok, a_tok, m_flat,
                species_table, items_table, abilities_table, moves_table,
                pre_species, pre_items, pre_abilities, pre_moves)


def _tc_combine(embsum, msum, preS, preI, preA, pmsum,
                s_tok2, i_tok2, a_tok2, m_tok, nm2,
                Ws, Wi, Wa, Wm, r0s, r0i, r0a, r0m):
    B, D = embsum.shape
    P = preS.shape[1]
    BLK = min(1024, B)
    f32 = jnp.float32

    def body(emb_r, msum_r, ps_r, pi_r, pa_r, pm_r,
             st_r, it_r, at_r, mt_r, nm_r,
             ws_r, wi_r, wa_r, wm_r, r0s_r, r0i_r, r0a_r, r0m_r, out_r):
        ws = ws_r[...]
        wi = wi_r[...]
        wa = wa_r[...]
        wm = wm_r[...]
        cs = jnp.dot(r0s_r[...], ws, preferred_element_type=f32)
        ci = jnp.dot(r0i_r[...], wi, preferred_element_type=f32)
        ca = jnp.dot(r0a_r[...], wa, preferred_element_type=f32)
        cm = jnp.dot(r0m_r[...], wm, preferred_element_type=f32)
        zs = (st_r[...] == 0).astype(f32)
        zi = (it_r[...] == 0).astype(f32)
        za = (at_r[...] == 0).astype(f32)
        cnt0 = jnp.sum((mt_r[...] == 0).astype(f32), axis=1, keepdims=True)
        nmf = jnp.maximum(nm_r[...], 1).astype(f32)
        lin = (jnp.dot(ps_r[...], ws, preferred_element_type=f32) - zs * cs
               + jnp.dot(pi_r[...], wi, preferred_element_type=f32) - zi * ci
               + jnp.dot(pa_r[...], wa, preferred_element_type=f32) - za * ca)
        mv = (jnp.dot(pm_r[...], wm, preferred_element_type=f32) - cnt0 * cm)
        out_r[...] = emb_r[...] + (msum_r[...] + mv) / nmf + lin

    blk_bd = pl.BlockSpec((BLK, D), lambda i: (i, 0))
    blk_bp = pl.BlockSpec((BLK, P), lambda i: (i, 0))
    blk_b1 = pl.BlockSpec((BLK, 1), lambda i: (i, 0))
    blk_b4 = pl.BlockSpec((BLK, 4), lambda i: (i, 0))
    blk_w = pl.BlockSpec((P, D), lambda i: (0, 0))
    blk_r0 = pl.BlockSpec((1, P), lambda i: (0, 0))

    return pl.pallas_call(
        body,
        grid=(B // BLK,),
        in_specs=[blk_bd, blk_bd, blk_bp, blk_bp, blk_bp, blk_bp,
                  blk_b1, blk_b1, blk_b1, blk_b4, blk_b1,
                  blk_w, blk_w, blk_w, blk_w,
                  blk_r0, blk_r0, blk_r0, blk_r0],
        out_specs=blk_bd,
        out_shape=jax.ShapeDtypeStruct((B, D), f32),
    )(embsum, msum, preS, preI, preA, pmsum,
      s_tok2, i_tok2, a_tok2, m_tok, nm2,
      Ws, Wi, Wa, Wm, r0s, r0i, r0a, r0m)


def kernel(species_tokens, ability_tokens, item_tokens, move_tokens, num_moves,
           species_table, items_table, abilities_table, moves_table,
           pre_species, pre_items, pre_abilities, pre_moves,
           species_W, items_W, abilities_W, moves_W):
    m_flat = move_tokens.reshape(-1)
    embsum, msum, preS, preI, preA, pmsum = _sc_gather_stage(
        species_tokens, item_tokens, ability_tokens, m_flat,
        species_table, items_table, abilities_table, moves_table,
        pre_species, pre_items, pre_abilities, pre_moves)
    return _tc_combine(
        embsum, msum, preS, preI, preA, pmsum,
        species_tokens[:, None], item_tokens[:, None], ability_tokens[:, None],
        move_tokens, num_moves[:, None],
        species_W, items_W, abilities_W, moves_W,
        pre_species[0:1], pre_items[0:1], pre_abilities[0:1], pre_moves[0:1])
